# baseline (speedup 1.0000x reference)
import functools

import jax
import jax.numpy as jnp
from jax import lax
from jax.experimental import pallas as pl
from jax.experimental.layout import Layout, with_layout_constraint
from jax.experimental.pallas import tpu as pltpu
from jax.experimental.pallas import tpu_sc as plsc

B = 16384
EMB = 32
NROW = 1000000
CHUNK = 128
NCHUNK = B // CHUNK


def _sc_gather_one(idx1d, tbl_w):
    info = plsc.get_sparse_core_info()
    nw = info.num_cores * info.num_subcores
    spw = B // nw
    nd2 = EMB // 2

    ntile = -(-NROW // 128)
    bandstride = ntile * 1024

    mesh = plsc.VectorSubcoreMesh(core_axis_name="c", subcore_axis_name="s")
    scratch = [
        pltpu.VMEM((spw,), jnp.int32),
        pltpu.VMEM((nd2, spw), jnp.int32),
        pltpu.VMEM((nd2, spw), jnp.int32),
        pltpu.VMEM((EMB, spw), jnp.float32),
        pltpu.SemaphoreType.DMA,
    ]

    @functools.partial(
        pl.kernel, mesh=mesh,
        out_type=jax.ShapeDtypeStruct((EMB, B), jnp.float32),
        scratch_types=scratch,
        compiler_params=pltpu.CompilerParams(
            use_tc_tiling_on_sc=False, needs_layout_passes=False))
    def body(idx_hbm, tbl_hbm, out, iv, off, wbuf, fbuf, sem):
        wid = lax.axis_index("s") * info.num_cores + lax.axis_index("c")
        s_base = wid * spw
        pltpu.sync_copy(idx_hbm.at[pl.ds(s_base, spw)], iv)

        def vec_body(k, _):
            s0 = pl.multiple_of(k * 16, 16)
            r = iv[pl.ds(s0, 16)]
            roff = ((r >> 7) << 10) + (r & 127)
            for d2 in range(nd2):
                base_d2 = (d2 // 8) * bandstride + (d2 % 8) * 128
                off[d2, pl.ds(s0, 16)] = roff + base_d2
            return 0

        lax.fori_loop(0, spw // 16, vec_body, 0)

        copies = [
            pltpu.async_copy(tbl_hbm.at[0].at[off.at[d2]], wbuf.at[d2], sem)
            for d2 in range(nd2)
        ]
        for c in copies:
            c.wait()

        def unpack_body(k, _):
            s0 = pl.multiple_of(k * 16, 16)
            for d2 in range(nd2):
                w = wbuf[d2, pl.ds(s0, 16)]
                fbuf[2 * d2, pl.ds(s0, 16)] = plsc.bitcast(
                    w << 16, jnp.float32)
                fbuf[2 * d2 + 1, pl.ds(s0, 16)] = plsc.bitcast(
                    w & jnp.int32(-65536), jnp.float32)
            return 0

        lax.fori_loop(0, spw // 16, unpack_body, 0)
        pltpu.sync_copy(fbuf, out.at[:, pl.ds(s_base, spw)])

    return body(idx1d, tbl_w)


def _dense_body(gu_r, gi_r, mu_r, mi_r, w0at_r, w0bt_r, b0_r, w1t_r, b1_r,
                w2t_r, b2_r, wpa_r, wpb_r, bp_r, out_r):
    nc = gu_r.shape[1]

    def cat(ref):
        return jnp.concatenate([ref[:, c, :] for c in range(nc)], axis=1)

    g = cat(gu_r) * cat(gi_r)
    h = jnp.dot(w0at_r[...], cat(mu_r), preferred_element_type=jnp.float32)
    h = h + jnp.dot(w0bt_r[...], cat(mi_r),
                    preferred_element_type=jnp.float32)
    h = jnp.maximum(h + b0_r[...], 0.0)
    h = jnp.dot(w1t_r[...], h, preferred_element_type=jnp.float32) + b1_r[...]
    h = jnp.maximum(h, 0.0)
    h = jnp.dot(w2t_r[...], h, preferred_element_type=jnp.float32) + b2_r[...]
    h = jnp.maximum(h, 0.0)
    o = (jnp.sum(g * wpa_r[...], axis=0) + jnp.sum(h * wpb_r[...], axis=0)
         + bp_r[0, 0])
    out_r[...] = o


def _tc_dense(gu3, gi3, mu3, mi3, w0at, w0bt, b0c, w1t, b1c, w2t, b2c,
              wpa, wpb, bp2, interpret=False):
    cpb = 16
    grid = (NCHUNK // cpb,)
    data_spec = pl.BlockSpec((EMB, cpb, CHUNK), lambda i: (0, i, 0))

    def full(shape):
        return pl.BlockSpec(shape, lambda i: tuple(0 for _ in shape))

    return pl.pallas_call(
        _dense_body,
        grid=grid,
        in_specs=[
            data_spec, data_spec, data_spec, data_spec,
            full((64, EMB)), full((64, EMB)), full((64, 1)),
            full((32, 64)), full((32, 1)),
            full((16, 32)), full((16, 1)),
            full((EMB, 1)), full((16, 1)), full((1, 1)),
        ],
        out_specs=pl.BlockSpec((cpb * CHUNK,), lambda i: (i,)),
        out_shape=jax.ShapeDtypeStruct((B,), jnp.float32),
        interpret=interpret,
    )(gu3, gi3, mu3, mi3, w0at, w0bt, b0c, w1t, b1c, w2t, b2c, wpa, wpb, bp2)


def kernel(user_indices, item_indices, gmf_user_table, gmf_item_table,
           mlp_user_table, mlp_item_table, W0, b0, W1, b1, W2, b2, Wp, bp):
    def tview(t):
        tw = lax.bitcast_convert_type(
            t.astype(jnp.bfloat16).reshape(NROW, EMB // 2, 2), jnp.int32)
        return with_layout_constraint(
            tw.T, Layout(major_to_minor=(0, 1), tiling=((8, 128),)))

    uidx = user_indices.astype(jnp.int32)
    iidx = item_indices.astype(jnp.int32)
    outs = [_sc_gather_one(idx, tview(tbl))
            for idx, tbl in ((uidx, gmf_user_table), (iidx, gmf_item_table),
                             (uidx, mlp_user_table), (iidx, mlp_item_table))]
    gu3, gi3, mu3, mi3 = (o.reshape(EMB, NCHUNK, CHUNK) for o in outs)
    return _tc_dense(gu3, gi3, mu3, mi3,
                     W0[:EMB].T, W0[EMB:].T, b0.reshape(-1, 1),
                     W1.T, b1.reshape(-1, 1), W2.T, b2.reshape(-1, 1),
                     Wp[:EMB], Wp[EMB:], bp.reshape(1, 1))


# speedup vs baseline: 5.9427x; 5.9427x over previous
import functools

import jax
import jax.numpy as jnp
from jax import lax
from jax.experimental import pallas as pl
from jax.experimental.layout import Layout, with_layout_constraint
from jax.experimental.pallas import tpu as pltpu
from jax.experimental.pallas import tpu_sc as plsc

B = 16384
EMB = 32
NROW = 1000000
CHUNK = 128
NCHUNK = B // CHUNK


def _sc_gather_one(idx1d, tbl_t):
    info = plsc.get_sparse_core_info()
    nw = info.num_cores * info.num_subcores
    spw = B // nw

    ntile = -(-NROW // 128)
    bandstride = ntile * 1024

    mesh = plsc.VectorSubcoreMesh(core_axis_name="c", subcore_axis_name="s")
    scratch = [
        pltpu.VMEM((spw,), jnp.int32),
        pltpu.VMEM((EMB, spw), jnp.int32),
        pltpu.VMEM((EMB, spw), jnp.float32),
        pltpu.SemaphoreType.DMA,
    ]

    @functools.partial(
        pl.kernel, mesh=mesh,
        out_type=jax.ShapeDtypeStruct((EMB, B), jnp.float32),
        scratch_types=scratch,
        compiler_params=pltpu.CompilerParams(
            use_tc_tiling_on_sc=False, needs_layout_passes=False))
    def body(idx_hbm, tbl_hbm, out, iv, off, buf, sem):
        wid = lax.axis_index("s") * info.num_cores + lax.axis_index("c")
        s_base = wid * spw
        pltpu.sync_copy(idx_hbm.at[pl.ds(s_base, spw)], iv)

        def vec_body(k, _):
            s0 = pl.multiple_of(k * 16, 16)
            r = iv[pl.ds(s0, 16)]
            roff = ((r >> 7) << 10) + (r & 127)
            for d in range(EMB):
                base_d = (d // 8) * bandstride + (d % 8) * 128
                off[d, pl.ds(s0, 16)] = roff + base_d
            return 0

        lax.fori_loop(0, spw // 16, vec_body, 0)

        copies = [
            pltpu.async_copy(tbl_hbm.at[0].at[off.at[d]], buf.at[d], sem)
            for d in range(EMB)
        ]
        for c in copies:
            c.wait()
        pltpu.sync_copy(buf, out.at[:, pl.ds(s_base, spw)])

    return body(idx1d, tbl_t)


def _dense_body(gu_r, gi_r, mu_r, mi_r, w0at_r, w0bt_r, b0_r, w1t_r, b1_r,
                w2t_r, b2_r, wpa_r, wpb_r, bp_r, out_r):
    nc = gu_r.shape[1]

    def cat(ref):
        return jnp.concatenate([ref[:, c, :] for c in range(nc)], axis=1)

    g = cat(gu_r) * cat(gi_r)
    h = jnp.dot(w0at_r[...], cat(mu_r), preferred_element_type=jnp.float32)
    h = h + jnp.dot(w0bt_r[...], cat(mi_r),
                    preferred_element_type=jnp.float32)
    h = jnp.maximum(h + b0_r[...], 0.0)
    h = jnp.dot(w1t_r[...], h, preferred_element_type=jnp.float32) + b1_r[...]
    h = jnp.maximum(h, 0.0)
    h = jnp.dot(w2t_r[...], h, preferred_element_type=jnp.float32) + b2_r[...]
    h = jnp.maximum(h, 0.0)
    o = (jnp.sum(g * wpa_r[...], axis=0) + jnp.sum(h * wpb_r[...], axis=0)
         + bp_r[0, 0])
    out_r[...] = o


def _tc_dense(gu3, gi3, mu3, mi3, w0at, w0bt, b0c, w1t, b1c, w2t, b2c,
              wpa, wpb, bp2, interpret=False):
    cpb = 16
    grid = (NCHUNK // cpb,)
    data_spec = pl.BlockSpec((EMB, cpb, CHUNK), lambda i: (0, i, 0))

    def full(shape):
        return pl.BlockSpec(shape, lambda i: tuple(0 for _ in shape))

    return pl.pallas_call(
        _dense_body,
        grid=grid,
        in_specs=[
            data_spec, data_spec, data_spec, data_spec,
            full((64, EMB)), full((64, EMB)), full((64, 1)),
            full((32, 64)), full((32, 1)),
            full((16, 32)), full((16, 1)),
            full((EMB, 1)), full((16, 1)), full((1, 1)),
        ],
        out_specs=pl.BlockSpec((cpb * CHUNK,), lambda i: (i,)),
        out_shape=jax.ShapeDtypeStruct((B,), jnp.float32),
        interpret=interpret,
    )(gu3, gi3, mu3, mi3, w0at, w0bt, b0c, w1t, b1c, w2t, b2c, wpa, wpb, bp2)


def kernel(user_indices, item_indices, gmf_user_table, gmf_item_table,
           mlp_user_table, mlp_item_table, W0, b0, W1, b1, W2, b2, Wp, bp):
    def tview(t):
        return with_layout_constraint(
            t.T, Layout(major_to_minor=(0, 1), tiling=((8, 128),)))

    uidx = user_indices.astype(jnp.int32)
    iidx = item_indices.astype(jnp.int32)
    outs = [_sc_gather_one(idx, tview(tbl))
            for idx, tbl in ((uidx, gmf_user_table), (iidx, gmf_item_table),
                             (uidx, mlp_user_table), (iidx, mlp_item_table))]
    gu3, gi3, mu3, mi3 = (o.reshape(EMB, NCHUNK, CHUNK) for o in outs)
    return _tc_dense(gu3, gi3, mu3, mi3,
                     W0[:EMB].T, W0[EMB:].T, b0.reshape(-1, 1),
                     W1.T, b1.reshape(-1, 1), W2.T, b2.reshape(-1, 1),
                     Wp[:EMB], Wp[EMB:], bp.reshape(1, 1))
